# baseline (device time: 22715 ns/iter reference)
import jax
import jax.numpy as jnp
from jax import lax
from jax.experimental import pallas as pl
from jax.experimental.pallas import tpu as pltpu

N_DEV = 16
WAVES = 2


def kernel(A, B):
    m, k = A.shape
    k2, n = B.shape
    rows = m // N_DEV
    half = rows // WAVES

    def body(a_ref, b_ref, out_ref, send16, rs_buf, ag_buf,
             rs_send, rs_recv, ag_send, ag_recv):
        d = lax.axis_index("i")

        def cw(c, w):
            return pl.ds(lax.rem(c + 2 * N_DEV, N_DEV) * rows + w * half, half)

        barrier_sem = pltpu.get_barrier_semaphore()
        for off in range(1, N_DEV):
            pl.semaphore_signal(
                barrier_sem, inc=1,
                device_id=(lax.rem(d + off, N_DEV),),
                device_id_type=pl.DeviceIdType.MESH,
            )

        out_ref[:, :] = jnp.dot(
            a_ref[:, :], b_ref[:, :], preferred_element_type=jnp.float32
        )

        rs_rdmas = []
        ag_rdmas = []

        def rs_cast(w):
            for off in range(1, N_DEV):
                o = lax.rem(d + off, N_DEV)
                send16[w, off] = out_ref[cw(o, w)].astype(jnp.bfloat16)

        def rs_send_wave(w):
            for off in range(1, N_DEV):
                o = lax.rem(d + off, N_DEV)
                rdma = pltpu.make_async_remote_copy(
                    src_ref=send16.at[w, off],
                    dst_ref=rs_buf.at[w, d],
                    send_sem=rs_send.at[w, o],
                    recv_sem=rs_recv.at[w, d],
                    device_id=(o,),
                    device_id_type=pl.DeviceIdType.MESH,
                )
                rdma.start()
                rs_rdmas.append(rdma)

        def reduce_silu_ag(w):
            acc = out_ref[cw(d, w)]
            for off in range(1, N_DEV):
                src = lax.rem(d + off, N_DEV)
                recv = pltpu.make_async_remote_copy(
                    src_ref=rs_buf.at[w, src],
                    dst_ref=rs_buf.at[w, src],
                    send_sem=rs_send.at[w, src],
                    recv_sem=rs_recv.at[w, src],
                    device_id=(src,),
                    device_id_type=pl.DeviceIdType.MESH,
                )
                recv.wait_recv()
                acc = acc + rs_buf[w, src].astype(jnp.float32)
            z = acc / (1.0 + jnp.exp(-acc))
            out_ref[cw(d, w)] = z
            send16[w, 0] = z.astype(jnp.bfloat16)
            for off in range(1, N_DEV):
                o = lax.rem(d + off, N_DEV)
                rdma = pltpu.make_async_remote_copy(
                    src_ref=send16.at[w, 0],
                    dst_ref=ag_buf.at[w, d],
                    send_sem=ag_send.at[w, o],
                    recv_sem=ag_recv.at[w, d],
                    device_id=(o,),
                    device_id_type=pl.DeviceIdType.MESH,
                )
                rdma.start()
                ag_rdmas.append(rdma)

        def ag_join(w):
            for off in range(1, N_DEV):
                src = lax.rem(d + off, N_DEV)
                recv = pltpu.make_async_remote_copy(
                    src_ref=ag_buf.at[w, src],
                    dst_ref=ag_buf.at[w, src],
                    send_sem=ag_send.at[w, src],
                    recv_sem=ag_recv.at[w, src],
                    device_id=(src,),
                    device_id_type=pl.DeviceIdType.MESH,
                )
                recv.wait_recv()
                out_ref[cw(src, w)] = ag_buf[w, src].astype(jnp.float32)

        rs_cast(0)
        pl.semaphore_wait(barrier_sem, N_DEV - 1)
        rs_send_wave(0)
        rs_cast(1)
        rs_send_wave(1)
        reduce_silu_ag(0)
        reduce_silu_ag(1)
        ag_join(0)
        ag_join(1)

        for rdma in rs_rdmas:
            rdma.wait_send()
        for rdma in ag_rdmas:
            rdma.wait_send()

    return pl.pallas_call(
        body,
        out_shape=jax.ShapeDtypeStruct((m, n), jnp.float32),
        in_specs=[
            pl.BlockSpec(memory_space=pltpu.VMEM),
            pl.BlockSpec(memory_space=pltpu.VMEM),
        ],
        out_specs=pl.BlockSpec(memory_space=pltpu.VMEM),
        scratch_shapes=[
            pltpu.VMEM((WAVES, N_DEV, half, n), jnp.bfloat16),
            pltpu.VMEM((WAVES, N_DEV, half, n), jnp.bfloat16),
            pltpu.VMEM((WAVES, N_DEV, half, n), jnp.bfloat16),
            pltpu.SemaphoreType.DMA((WAVES, N_DEV)),
            pltpu.SemaphoreType.DMA((WAVES, N_DEV)),
            pltpu.SemaphoreType.DMA((WAVES, N_DEV)),
            pltpu.SemaphoreType.DMA((WAVES, N_DEV)),
        ],
        compiler_params=pltpu.CompilerParams(collective_id=0),
    )(A, B)


# device time: 22503 ns/iter; 1.0094x vs baseline; 1.0094x over previous
import jax
import jax.numpy as jnp
from jax import lax
from jax.experimental import pallas as pl
from jax.experimental.pallas import tpu as pltpu

N_DEV = 16


def kernel(A, B):
    m, k = A.shape
    k2, n = B.shape
    rows = m // N_DEV

    def body(a_ref, b_ref, out_ref, send16, rs_buf, ag_buf,
             rs_send, rs_recv, ag_send, ag_recv):
        d = lax.axis_index("i")

        def chunk(c):
            return pl.ds(lax.rem(c + 2 * N_DEV, N_DEV) * rows, rows)

        barrier_sem = pltpu.get_barrier_semaphore()
        for off in range(1, N_DEV):
            pl.semaphore_signal(
                barrier_sem, inc=1,
                device_id=(lax.rem(d + off, N_DEV),),
                device_id_type=pl.DeviceIdType.MESH,
            )

        out_ref[:, :] = jnp.dot(
            a_ref[:, :], b_ref[:, :], preferred_element_type=jnp.float32
        )

        for off in range(1, N_DEV):
            o = lax.rem(d + off, N_DEV)
            send16[off] = out_ref[chunk(o)].astype(jnp.bfloat16)

        pl.semaphore_wait(barrier_sem, N_DEV - 1)

        rs_rdmas = []
        for off in range(1, N_DEV):
            o = lax.rem(d + off, N_DEV)
            rdma = pltpu.make_async_remote_copy(
                src_ref=send16.at[off],
                dst_ref=rs_buf.at[d],
                send_sem=rs_send.at[o],
                recv_sem=rs_recv.at[d],
                device_id=(o,),
                device_id_type=pl.DeviceIdType.MESH,
            )
            rdma.start()
            rs_rdmas.append(rdma)

        acc = out_ref[chunk(d)]
        for off in range(1, N_DEV):
            src = lax.rem(d + off, N_DEV)
            recv = pltpu.make_async_remote_copy(
                src_ref=rs_buf.at[src],
                dst_ref=rs_buf.at[src],
                send_sem=rs_send.at[src],
                recv_sem=rs_recv.at[src],
                device_id=(src,),
                device_id_type=pl.DeviceIdType.MESH,
            )
            recv.wait_recv()
            acc = acc + rs_buf[src].astype(jnp.float32)

        z = acc / (1.0 + jnp.exp(-acc))
        out_ref[chunk(d)] = z
        send16[0] = z.astype(jnp.bfloat16)

        ag_rdmas = []
        for off in range(1, N_DEV):
            o = lax.rem(d + off, N_DEV)
            rdma = pltpu.make_async_remote_copy(
                src_ref=send16.at[0],
                dst_ref=ag_buf.at[d],
                send_sem=ag_send.at[o],
                recv_sem=ag_recv.at[d],
                device_id=(o,),
                device_id_type=pl.DeviceIdType.MESH,
            )
            rdma.start()
            ag_rdmas.append(rdma)

        for off in range(1, N_DEV):
            src = lax.rem(d + off, N_DEV)
            recv = pltpu.make_async_remote_copy(
                src_ref=ag_buf.at[src],
                dst_ref=ag_buf.at[src],
                send_sem=ag_send.at[src],
                recv_sem=ag_recv.at[src],
                device_id=(src,),
                device_id_type=pl.DeviceIdType.MESH,
            )
            recv.wait_recv()
            out_ref[chunk(src)] = ag_buf[src].astype(jnp.float32)

        for rdma in rs_rdmas:
            rdma.wait_send()
        for rdma in ag_rdmas:
            rdma.wait_send()

    return pl.pallas_call(
        body,
        out_shape=jax.ShapeDtypeStruct((m, n), jnp.float32),
        in_specs=[
            pl.BlockSpec(memory_space=pltpu.VMEM),
            pl.BlockSpec(memory_space=pltpu.VMEM),
        ],
        out_specs=pl.BlockSpec(memory_space=pltpu.VMEM),
        scratch_shapes=[
            pltpu.VMEM((N_DEV, m // N_DEV, n), jnp.bfloat16),
            pltpu.VMEM((N_DEV, m // N_DEV, n), jnp.bfloat16),
            pltpu.VMEM((N_DEV, m // N_DEV, n), jnp.bfloat16),
            pltpu.SemaphoreType.DMA((N_DEV,)),
            pltpu.SemaphoreType.DMA((N_DEV,)),
            pltpu.SemaphoreType.DMA((N_DEV,)),
            pltpu.SemaphoreType.DMA((N_DEV,)),
        ],
        compiler_params=pltpu.CompilerParams(collective_id=0),
    )(A, B)
